# baseline (speedup 1.0000x reference)
import jax
import jax.numpy as jnp
from jax.experimental import pallas as pl

INPUT_DIM = 832
HIDDEN1 = 256
HIDDEN2 = 32
BATCH = 16384
BN = 4096


def _mlp_block(xt_ref, w1t_ref, b1_ref, w2_ref, b2_ref, w3_ref, b3_ref, out_ref):
    b1c = jnp.transpose(b1_ref[...])
    b2c = jnp.transpose(b2_ref[...])
    w3c = jnp.transpose(w3_ref[...])
    xt = xt_ref[...]
    h = jax.lax.dot_general(
        w1t_ref[...], xt, (((0,), (0,)), ((), ())),
        preferred_element_type=jnp.float32)
    h = jnp.maximum(h + b1c, 0.0)
    h = jax.lax.dot_general(
        w2_ref[...].astype(jnp.bfloat16), h.astype(jnp.bfloat16),
        (((1,), (0,)), ((), ())),
        preferred_element_type=jnp.float32)
    h = jnp.maximum(h + b2c, 0.0)
    out = jnp.sum(h * w3c, axis=0, keepdims=True) + b3_ref[0, 0]
    out_ref[...] = out


def kernel(x, w1, b1, w2, b2, w3, b3):
    xt = x.T
    w1t = w1.T
    b1r = b1.reshape(1, HIDDEN1)
    b2r = b2.reshape(1, HIDDEN2)
    b3r = b3.reshape(1, 1)

    grid = (BATCH // BN,)
    const = lambda i: (0, 0)
    outt = pl.pallas_call(
        _mlp_block,
        grid=grid,
        in_specs=[
            pl.BlockSpec((INPUT_DIM, BN), lambda i: (0, i)),
            pl.BlockSpec((INPUT_DIM, HIDDEN1), const),
            pl.BlockSpec((1, HIDDEN1), const),
            pl.BlockSpec((HIDDEN2, HIDDEN1), const),
            pl.BlockSpec((1, HIDDEN2), const),
            pl.BlockSpec((1, HIDDEN2), const),
            pl.BlockSpec((1, 1), const),
        ],
        out_specs=pl.BlockSpec((1, BN), lambda i: (0, i)),
        out_shape=jax.ShapeDtypeStruct((1, BATCH), jnp.float32),
    )(xt, w1t, b1r, w2, b2r, w3, b3r)
    return outt.reshape(BATCH, 1)


# speedup vs baseline: 1.0064x; 1.0064x over previous
import jax
import jax.numpy as jnp
from jax.experimental import pallas as pl

INPUT_DIM = 832
HIDDEN1 = 256
HIDDEN2 = 32
BATCH = 16384
BN = 4096


def _mlp_block(xt_ref, w1t_ref, b1_ref, w2_ref, b2_ref, w3_ref, b3_ref, out_ref):
    b1c = jnp.transpose(b1_ref[...])
    b2c = jnp.transpose(b2_ref[...])
    w3c = jnp.transpose(w3_ref[...])
    xt = xt_ref[...].astype(jnp.bfloat16)
    h = jax.lax.dot_general(
        w1t_ref[...].astype(jnp.bfloat16), xt, (((0,), (0,)), ((), ())),
        preferred_element_type=jnp.float32)
    h = jnp.maximum(h + b1c, 0.0)
    h = jax.lax.dot_general(
        w2_ref[...], h,
        (((1,), (0,)), ((), ())),
        preferred_element_type=jnp.float32)
    h = jnp.maximum(h + b2c, 0.0)
    out = jnp.sum(h * w3c, axis=0, keepdims=True) + b3_ref[0, 0]
    out_ref[...] = out


def kernel(x, w1, b1, w2, b2, w3, b3):
    xt = x.T
    w1t = w1.T
    b1r = b1.reshape(1, HIDDEN1)
    b2r = b2.reshape(1, HIDDEN2)
    b3r = b3.reshape(1, 1)

    grid = (BATCH // BN,)
    const = lambda i: (0, 0)
    outt = pl.pallas_call(
        _mlp_block,
        grid=grid,
        in_specs=[
            pl.BlockSpec((INPUT_DIM, BN), lambda i: (0, i)),
            pl.BlockSpec((INPUT_DIM, HIDDEN1), const),
            pl.BlockSpec((1, HIDDEN1), const),
            pl.BlockSpec((HIDDEN2, HIDDEN1), const),
            pl.BlockSpec((1, HIDDEN2), const),
            pl.BlockSpec((1, HIDDEN2), const),
            pl.BlockSpec((1, 1), const),
        ],
        out_specs=pl.BlockSpec((1, BN), lambda i: (0, i)),
        out_shape=jax.ShapeDtypeStruct((1, BATCH), jnp.float32),
    )(xt, w1t, b1r, w2, b2r, w3, b3r)
    return outt.reshape(BATCH, 1)
